# baseline (device time: 21382 ns/iter reference)
import jax
import jax.numpy as jnp
from jax import lax
from jax.experimental import pallas as pl
from jax.experimental.pallas import tpu as pltpu

C = 32


def kernel(x):
    _, m, n_shard = x.shape
    n_full = 2 * n_shard
    mc = m // C

    def body(x_ref, out_ref, comm_ref, xs_sems, xr_sems, ys_sems, yr_sems):
        my_x = lax.axis_index("x")
        my_y = lax.axis_index("y")
        x_peer = (1 - my_x, my_y)
        y_peer = (my_x, 1 - my_y)
        my_col = my_y * n_shard

        barrier_sem = pltpu.get_barrier_semaphore()
        for nbr in [x_peer, y_peer]:
            pl.semaphore_signal(
                barrier_sem, inc=1,
                device_id=nbr, device_id_type=pl.DeviceIdType.MESH,
            )
        pl.semaphore_wait(barrier_sem, 2)

        rdma_x = []
        for c in range(C):
            r = pltpu.make_async_remote_copy(
                src_ref=x_ref.at[0, pl.ds(c * mc, mc), :],
                dst_ref=comm_ref.at[pl.ds(c * mc, mc), :],
                send_sem=xs_sems.at[c],
                recv_sem=xr_sems.at[c],
                device_id=x_peer,
                device_id_type=pl.DeviceIdType.MESH,
            )
            r.start()
            rdma_x.append(r)

        rdma_y = []
        for c in range(C):
            rdma_x[c].wait_recv()
            rows = pl.ds(c * mc, mc)
            out_ref[rows, pl.ds(my_col, n_shard)] = (
                x_ref[0, rows, :] + comm_ref[rows, :]
            )
            r = pltpu.make_async_remote_copy(
                src_ref=out_ref.at[rows, pl.ds(my_col, n_shard)],
                dst_ref=out_ref.at[rows, pl.ds(my_col, n_shard)],
                send_sem=ys_sems.at[c],
                recv_sem=yr_sems.at[c],
                device_id=y_peer,
                device_id_type=pl.DeviceIdType.MESH,
            )
            r.start()
            rdma_y.append(r)

        for c in range(C):
            rdma_y[c].wait_recv()
        for c in range(C):
            rdma_x[c].wait_send()
            rdma_y[c].wait_send()

    return pl.pallas_call(
        body,
        out_shape=jax.ShapeDtypeStruct((m, n_full), x.dtype),
        in_specs=[pl.BlockSpec(memory_space=pltpu.VMEM)],
        out_specs=pl.BlockSpec(memory_space=pltpu.VMEM),
        scratch_shapes=[
            pltpu.VMEM((m, n_shard), x.dtype),
            pltpu.SemaphoreType.DMA((C,)),
            pltpu.SemaphoreType.DMA((C,)),
            pltpu.SemaphoreType.DMA((C,)),
            pltpu.SemaphoreType.DMA((C,)),
        ],
        compiler_params=pltpu.CompilerParams(collective_id=0),
    )(x)


# device time: 21038 ns/iter; 1.0164x vs baseline; 1.0164x over previous
import jax
import jax.numpy as jnp
from jax import lax
from jax.experimental import pallas as pl
from jax.experimental.pallas import tpu as pltpu

C = 16


def kernel(x):
    _, m, n_shard = x.shape
    n_full = 2 * n_shard
    mc = m // C

    def body(x_ref, out_ref, comm_ref, xs_sems, xr_sems, ys_sems, yr_sems):
        my_x = lax.axis_index("x")
        my_y = lax.axis_index("y")
        x_peer = (1 - my_x, my_y)
        y_peer = (my_x, 1 - my_y)
        my_col = my_y * n_shard

        barrier_sem = pltpu.get_barrier_semaphore()
        for nbr in [x_peer, y_peer]:
            pl.semaphore_signal(
                barrier_sem, inc=1,
                device_id=nbr, device_id_type=pl.DeviceIdType.MESH,
            )
        pl.semaphore_wait(barrier_sem, 2)

        rdma_x = []
        for c in range(C):
            r = pltpu.make_async_remote_copy(
                src_ref=x_ref.at[0, pl.ds(c * mc, mc), :],
                dst_ref=comm_ref.at[pl.ds(c * mc, mc), :],
                send_sem=xs_sems.at[c],
                recv_sem=xr_sems.at[c],
                device_id=x_peer,
                device_id_type=pl.DeviceIdType.MESH,
            )
            r.start()
            rdma_x.append(r)

        rdma_y = []
        for c in range(C):
            rdma_x[c].wait_recv()
            rows = pl.ds(c * mc, mc)
            out_ref[rows, pl.ds(my_col, n_shard)] = (
                x_ref[0, rows, :] + comm_ref[rows, :]
            )
            r = pltpu.make_async_remote_copy(
                src_ref=out_ref.at[rows, pl.ds(my_col, n_shard)],
                dst_ref=out_ref.at[rows, pl.ds(my_col, n_shard)],
                send_sem=ys_sems.at[c],
                recv_sem=yr_sems.at[c],
                device_id=y_peer,
                device_id_type=pl.DeviceIdType.MESH,
            )
            r.start()
            rdma_y.append(r)

        for c in range(C):
            rdma_y[c].wait_recv()
        for c in range(C):
            rdma_x[c].wait_send()
            rdma_y[c].wait_send()

    return pl.pallas_call(
        body,
        out_shape=jax.ShapeDtypeStruct((m, n_full), x.dtype),
        in_specs=[pl.BlockSpec(memory_space=pltpu.VMEM)],
        out_specs=pl.BlockSpec(memory_space=pltpu.VMEM),
        scratch_shapes=[
            pltpu.VMEM((m, n_shard), x.dtype),
            pltpu.SemaphoreType.DMA((C,)),
            pltpu.SemaphoreType.DMA((C,)),
            pltpu.SemaphoreType.DMA((C,)),
            pltpu.SemaphoreType.DMA((C,)),
        ],
        compiler_params=pltpu.CompilerParams(collective_id=0),
    )(x)
